# Initial kernel scaffold; baseline (speedup 1.0000x reference)
import jax
import jax.numpy as jnp
from jax.experimental import pallas as pl


def kernel(point_cloud, robot0_eef_pos, robot0_eef_quat, robot0_gripper_qpos, params):
    raise NotImplementedError("write your pallas kernel here")


# speedup vs baseline: 12.5468x; 12.5468x over previous
import functools

import jax
import jax.numpy as jnp
from jax.experimental import pallas as pl

B = 16
N0 = 1162
N1 = 581
N2 = 146
K = 64
BIG = 1e30


def _iota_like(x, axis):
    return jax.lax.broadcasted_iota(jnp.int32, x.shape, axis)


def _first_match_onehot(d, m):
    n = d.shape[-1]
    ii = _iota_like(d, d.ndim - 1)
    cand = jnp.where(d == m, ii, n)
    amin = jnp.min(cand, axis=-1, keepdims=True)
    return (ii == amin).astype(jnp.float32)


def _fps_kernel(post_ref, posq_ref, *, n):
    post = post_ref[...]
    nb, _, npts = post.shape
    pc = [post[:, c, :] for c in range(3)]
    last = [p[:, 0:1] for p in pc]
    d = jnp.full((nb, npts), 1e10, jnp.float32)
    posq_ref[:, 0:1, :] = jnp.concatenate(
        [l[:, :, None] for l in last], axis=-1)

    def body(i, st):
        d, l0, l1, l2 = st
        last = (l0, l1, l2)
        dn = sum((pc[c] - last[c]) ** 2 for c in range(3))
        d = jnp.minimum(d, dn)
        m = jnp.max(d, axis=-1, keepdims=True)
        oh = _first_match_onehot(d, m)
        nl = [jnp.sum(oh * pc[c], axis=-1, keepdims=True) for c in range(3)]
        posq_ref[:, pl.ds(i, 1), :] = jnp.concatenate(
            [l[:, :, None] for l in nl], axis=-1)
        return d, nl[0], nl[1], nl[2]

    jax.lax.fori_loop(1, n, body, (d, last[0], last[1], last[2]))


def _fps(pos, n):
    nb, npts, _ = pos.shape
    post = jnp.swapaxes(pos, 1, 2)
    return pl.pallas_call(
        functools.partial(_fps_kernel, n=n),
        grid=(1,),
        in_specs=[pl.BlockSpec((nb, 3, npts), lambda i: (0, 0, 0))],
        out_specs=pl.BlockSpec((nb, n, 3), lambda i: (0, 0, 0)),
        out_shape=jax.ShapeDtypeStruct((nb, n, 3), jnp.float32),
    )(post)


def _pair_d2(pq, pst):
    return ((pq[:, 0:1] - pst[0:1, :]) ** 2
            + (pq[:, 1:2] - pst[1:2, :]) ** 2
            + (pq[:, 2:3] - pst[2:3, :]) ** 2)


def _sa_kernel(x_ref, pos_ref, post_ref, posq_ref,
               w1_ref, b1_ref, w2_ref, b2_ref, w3_ref, b3_ref,
               out_ref, *, r2, cin):
    x = x_ref[0]
    pos = pos_ref[0]
    post = post_ref[0]
    posq = posq_ref[0]
    w1 = w1_ref[...]
    b1 = b1_ref[...]
    w2 = w2_ref[...]
    b2 = b2_ref[...]
    w3 = w3_ref[...]
    b3 = b3_ref[...]
    cout = w3.shape[1]
    nq = posq.shape[0]

    a = jnp.concatenate([x, pos], axis=-1) @ w1 + b1
    bq = posq @ w1[cin:, :]

    d2 = _pair_d2(posq, post)

    def body(j, st):
        d2c, acc = st
        m = jnp.min(d2c, axis=-1, keepdims=True)
        oh = _first_match_onehot(d2c, m)
        d2c = jnp.where(oh > 0.0, BIG, d2c)
        h = jnp.maximum(oh @ a - bq, 0.0)
        h = jnp.maximum(h @ w2 + b2, 0.0)
        msg = h @ w3 + b3
        msg = jnp.where(m <= r2, msg, -1e10)
        return d2c, jnp.maximum(acc, msg)

    acc0 = jnp.full((nq, cout), -1e10, jnp.float32)
    _, out = jax.lax.fori_loop(0, K, body, (d2, acc0))
    out_ref[0] = out


def _sa(x, pos, posq, p, r):
    nb, npts, cin = x.shape
    nq = posq.shape[1]
    (w1, b1), (w2, b2), (w3, b3) = p
    cout = w3.shape[1]
    b1 = b1.reshape(1, -1)
    b2 = b2.reshape(1, -1)
    b3 = b3.reshape(1, -1)
    wspec = lambda w: pl.BlockSpec(w.shape, lambda b: (0,) * w.ndim)
    return pl.pallas_call(
        functools.partial(_sa_kernel, r2=r * r, cin=cin),
        grid=(nb,),
        in_specs=[
            pl.BlockSpec((1, npts, cin), lambda b: (b, 0, 0)),
            pl.BlockSpec((1, npts, 3), lambda b: (b, 0, 0)),
            pl.BlockSpec((1, 3, npts), lambda b: (b, 0, 0)),
            pl.BlockSpec((1, nq, 3), lambda b: (b, 0, 0)),
            wspec(w1), wspec(b1), wspec(w2), wspec(b2), wspec(w3), wspec(b3),
        ],
        out_specs=pl.BlockSpec((1, nq, cout), lambda b: (b, 0, 0)),
        out_shape=jax.ShapeDtypeStruct((nb, nq, cout), jnp.float32),
    )(x, pos, jnp.swapaxes(pos, 1, 2), posq, w1, b1, w2, b2, w3, b3)


def _mlp(ws, h):
    n = len(ws)
    for i, (w, b) in enumerate(ws):
        h = h @ w + b
        if i < n - 1:
            h = jnp.maximum(h, 0.0)
    return h


def _knn3(xsrc, psrct, pq):
    d2 = _pair_d2(pq, psrct)
    num = jnp.zeros((pq.shape[0], xsrc.shape[1]), jnp.float32)
    den = jnp.zeros((pq.shape[0], 1), jnp.float32)
    for _ in range(3):
        m = jnp.min(d2, axis=-1, keepdims=True)
        oh = _first_match_onehot(d2, m)
        d2 = jnp.where(oh > 0.0, BIG, d2)
        w = 1.0 / jnp.maximum(m, 1e-16)
        num = num + w * (oh @ xsrc)
        den = den + w
    return num / den


def _tail_kernel(x2_ref, pos2_ref, p2t_ref, x1_ref, pos1_ref, p1t_ref,
                 x0_ref, pos0_ref, st_ref, *rest):
    refs = list(rest)
    flow_ref, fo_ref = refs[-2], refs[-1]
    wrefs = refs[:-2]

    def take(nlayers):
        nonlocal wrefs
        ws = []
        for _ in range(nlayers):
            w, b = wrefs[0], wrefs[1]
            wrefs = wrefs[2:]
            ws.append((w[...], b[...]))
        return ws

    sa3 = take(3)
    fp3 = take(2)
    fp2 = take(2)
    fp1 = take(3)
    head = take(3)
    state = take(2)
    finger = take(1)

    x2 = x2_ref[0]
    pos2 = pos2_ref[0]
    p2t = p2t_ref[0]
    x1 = x1_ref[0]
    pos1 = pos1_ref[0]
    p1t = p1t_ref[0]
    x0 = x0_ref[0]
    pos0 = pos0_ref[0]
    stv = st_ref[0]

    g = _mlp(sa3, jnp.concatenate([x2, pos2], -1))
    x3 = jnp.max(g, axis=0, keepdims=True)

    xi = jnp.broadcast_to(x3, (x2.shape[0], x3.shape[1]))
    x = _mlp(fp3, jnp.concatenate([xi, x2], -1))

    xi = _knn3(x, p2t, pos1)
    x = _mlp(fp2, jnp.concatenate([xi, x1], -1))

    xi = _knn3(x, p1t, pos0)
    x = _mlp(fp1, jnp.concatenate([xi, x0], -1))

    flow_ref[0] = _mlp(head, x)

    xmean = jnp.mean(x, axis=0, keepdims=True)
    sf = _mlp(state, stv)
    fo_ref[0] = _mlp(finger, jnp.concatenate([xmean, sf], -1))


def _tail(x2, pos2, x1, pos1, x0, pos0, stv, params):
    nb = x2.shape[0]
    p2t = jnp.swapaxes(pos2, 1, 2)
    p1t = jnp.swapaxes(pos1, 1, 2)
    wlist = []
    for name in ('sa3', 'fp3', 'fp2', 'fp1', 'head', 'state', 'finger'):
        for w, b in params[name]:
            wlist.append(w)
            wlist.append(b.reshape(1, -1))
    wspecs = [pl.BlockSpec(w.shape, lambda b: (0, 0)) for w in wlist]

    def bspec(arr):
        return pl.BlockSpec((1,) + arr.shape[1:], lambda b: (b, 0, 0))

    flow, fo = pl.pallas_call(
        _tail_kernel,
        grid=(nb,),
        in_specs=[bspec(x2), bspec(pos2), bspec(p2t), bspec(x1),
                  bspec(pos1), bspec(p1t), bspec(x0), bspec(pos0),
                  bspec(stv)] + wspecs,
        out_specs=[
            pl.BlockSpec((1, N0, 3), lambda b: (b, 0, 0)),
            pl.BlockSpec((1, 1, 1), lambda b: (b, 0, 0)),
        ],
        out_shape=[
            jax.ShapeDtypeStruct((nb, N0, 3), jnp.float32),
            jax.ShapeDtypeStruct((nb, 1, 1), jnp.float32),
        ],
    )(x2, pos2, p2t, x1, pos1, p1t, x0, pos0, stv, *wlist)
    return flow, fo


def kernel(point_cloud, robot0_eef_pos, robot0_eef_quat, robot0_gripper_qpos,
           params):
    pts = point_cloud.at[..., :1024, 3:].set(0.0)
    pts = pts.at[..., 1024:N0, 3:].set(
        pts[..., N0:, :3] - pts[..., 1024:N0, :3])
    pu = pts[..., :N0, :]
    x0 = pu[..., 3:]
    pos0 = pu[..., :3]

    pos1 = _fps(pos0, N1)
    x1 = _sa(x0, pos0, pos1, params['sa1'], 0.2)
    pos2 = _fps(pos1, N2)
    x2 = _sa(x1, pos1, pos2, params['sa2'], 0.4)

    stv = jnp.concatenate(
        [robot0_eef_pos, robot0_eef_quat, robot0_gripper_qpos], -1)
    stv = stv[:, None, :]

    flow, fo = _tail(x2, pos2, x1, pos1, x0, pos0, stv, params)
    nb = point_cloud.shape[0]
    return flow.reshape(nb, 1, N0, 3), fo.reshape(nb, 1)
